# baseline (speedup 1.0000x reference)
import functools

import jax
import jax.numpy as jnp
from jax import lax
from jax.experimental import pallas as pl
from jax.experimental.pallas import tpu as pltpu
from jax.experimental.pallas import tpu_sc as plsc

B = 4096
D = 64
TAG_LEN = 50
CAT_LEN = 20
L = 16
NC = 2
NS = 16
NW = NC * NS
BPW = B // NW

TBAGS = 2
CBAGS = 4
TG = BPW // TBAGS
CG = BPW // CBAGS
NBUF = 4

_mesh = plsc.VectorSubcoreMesh(core_axis_name="c", subcore_axis_name="s")


@functools.partial(
    pl.kernel,
    mesh=_mesh,
    compiler_params=pltpu.CompilerParams(use_tc_tiling_on_sc=False),
    out_type=(
        jax.ShapeDtypeStruct((B, D), jnp.float32),
        jax.ShapeDtypeStruct((B, D), jnp.float32),
    ),
    scratch_types=[
        pltpu.VMEM((TG, TBAGS * TAG_LEN), jnp.int32),
        pltpu.VMEM((CG, CBAGS * CAT_LEN), jnp.int32),
        pltpu.VMEM((NBUF, TBAGS * TAG_LEN, D), jnp.float32),
        pltpu.VMEM((BPW, D), jnp.float32),
        pltpu.VMEM((BPW, D), jnp.float32),
        pltpu.SemaphoreType.DMA,
        pltpu.SemaphoreType.DMA,
        pltpu.SemaphoreType.DMA,
        pltpu.SemaphoreType.DMA,
    ],
)
def _encode(tags_g, cats_g, tag_tab, cat_tab, out_t, out_c,
            tidx, cidx, rows, acc_t, acc_c, sem0, sem1, sem2, sem3):
    wid = lax.axis_index("s") * NC + lax.axis_index("c")
    b_base = wid * BPW
    sems = (sem0, sem1, sem2, sem3)

    pltpu.sync_copy(tags_g.at[pl.ds(wid * TG, TG)], tidx)
    pltpu.sync_copy(cats_g.at[pl.ds(wid * CG, CG)], cidx)

    def reduce_group(p, g, n_bags, bag_len, acc):
        for q in range(n_bags):
            b = g * n_bags + q
            r0 = q * bag_len
            for d in range(4):
                sl = pl.ds(d * L, L)
                v0 = rows[p, r0, sl]
                v1 = rows[p, r0 + 1, sl]
                for j in range(2, bag_len, 2):
                    v0 = v0 + rows[p, r0 + j, sl]
                    v1 = v1 + rows[p, r0 + j + 1, sl]
                acc[b, sl] = v0 + v1

    def run_phase(idx_ref, tab, n_groups, n_bags, bag_len, acc):
        dst = lambda p: rows.at[p, pl.ds(0, n_bags * bag_len)]

        def fire(g, p):
            pltpu.async_copy(tab.at[idx_ref.at[g]], dst(p), sems[p])

        def wait(p):
            pltpu.make_async_copy(tab.at[idx_ref.at[0]], dst(p), sems[p]).wait()

        for p in range(NBUF):
            fire(p, p)

        def body(gg, carry):
            for p in range(NBUF):
                g = NBUF * gg + p
                wait(p)
                reduce_group(p, g, n_bags, bag_len, acc)

                @pl.when(g + NBUF < n_groups)
                def _():
                    fire(g + NBUF, p)
            return carry

        lax.fori_loop(0, n_groups // NBUF, body, 0)

    run_phase(tidx, tag_tab, TG, TBAGS, TAG_LEN, acc_t)
    run_phase(cidx, cat_tab, CG, CBAGS, CAT_LEN, acc_c)

    pltpu.sync_copy(acc_t, out_t.at[pl.ds(b_base, BPW)])
    pltpu.sync_copy(acc_c, out_c.at[pl.ds(b_base, BPW)])


def kernel(tags, categories, tag_table, cat_table):
    tags_g = tags.reshape(B // TBAGS, TBAGS * TAG_LEN)
    cats_g = categories.reshape(B // CBAGS, CBAGS * CAT_LEN)
    return _encode(tags_g, cats_g, tag_table, cat_table)


# speedup vs baseline: 1.1468x; 1.1468x over previous
import functools

import jax
import jax.numpy as jnp
from jax import lax
from jax.experimental import pallas as pl
from jax.experimental.pallas import tpu as pltpu
from jax.experimental.pallas import tpu_sc as plsc

B = 4096
D = 64
TAG_LEN = 50
CAT_LEN = 20
L = 16
NC = 2
NS = 16
NW = NC * NS
BPW = B // NW

TBAGS = 2
CBAGS = 4
TG = BPW // TBAGS
CG = BPW // CBAGS
NBUF = 2

_mesh = plsc.VectorSubcoreMesh(core_axis_name="c", subcore_axis_name="s")


@functools.partial(
    pl.kernel,
    mesh=_mesh,
    compiler_params=pltpu.CompilerParams(use_tc_tiling_on_sc=False),
    out_type=(
        jax.ShapeDtypeStruct((B, D), jnp.float32),
        jax.ShapeDtypeStruct((B, D), jnp.float32),
    ),
    scratch_types=[
        pltpu.VMEM((TG, TBAGS * TAG_LEN), jnp.int32),
        pltpu.VMEM((CG, CBAGS * CAT_LEN), jnp.int32),
        pltpu.VMEM((NBUF, TBAGS * TAG_LEN, D), jnp.float32),
        pltpu.VMEM((BPW, D), jnp.float32),
        pltpu.VMEM((BPW, D), jnp.float32),
        pltpu.VMEM_SHARED((1000, D), jnp.float32),
        pltpu.SemaphoreType.DMA,
        pltpu.SemaphoreType.DMA,
    ],
)
def _encode(tags_g, cats_g, tag_tab, cat_tab, out_t, out_c,
            tidx, cidx, rows, acc_t, acc_c, cat_sp, sem0, sem1):
    wid = lax.axis_index("s") * NC + lax.axis_index("c")
    b_base = wid * BPW
    sems = (sem0, sem1)

    pltpu.sync_copy(tags_g.at[pl.ds(wid * TG, TG)], tidx)
    pltpu.sync_copy(cats_g.at[pl.ds(wid * CG, CG)], cidx)

    def reduce_group(p, g, n_bags, bag_len, acc):
        for q in range(n_bags):
            b = g * n_bags + q
            r0 = q * bag_len
            for d in range(4):
                sl = pl.ds(d * L, L)
                v0 = rows[p, r0, sl]
                v1 = rows[p, r0 + 1, sl]
                for j in range(2, bag_len, 2):
                    v0 = v0 + rows[p, r0 + j, sl]
                    v1 = v1 + rows[p, r0 + j + 1, sl]
                acc[b, sl] = v0 + v1

    def run_phase(idx_ref, tab, n_groups, n_bags, bag_len, acc):
        dst = lambda p: rows.at[p, pl.ds(0, n_bags * bag_len)]

        def fire(g, p):
            pltpu.async_copy(tab.at[idx_ref.at[g]], dst(p), sems[p])

        def wait(p):
            pltpu.make_async_copy(tab.at[idx_ref.at[0]], dst(p), sems[p]).wait()

        for p in range(NBUF):
            fire(p, p)

        def body(gg, carry):
            for p in range(NBUF):
                g = NBUF * gg + p
                wait(p)
                reduce_group(p, g, n_bags, bag_len, acc)

                @pl.when(g + NBUF < n_groups)
                def _():
                    fire(g + NBUF, p)
            return carry

        lax.fori_loop(0, n_groups // NBUF, body, 0)

    @pl.when(lax.axis_index("s") == 0)
    def _():
        pltpu.sync_copy(cat_tab, cat_sp)
    run_phase(tidx, tag_tab, TG, TBAGS, TAG_LEN, acc_t)
    plsc.subcore_barrier()
    run_phase(cidx, cat_sp, CG, CBAGS, CAT_LEN, acc_c)

    pltpu.sync_copy(acc_t, out_t.at[pl.ds(b_base, BPW)])
    pltpu.sync_copy(acc_c, out_c.at[pl.ds(b_base, BPW)])


def kernel(tags, categories, tag_table, cat_table):
    tags_g = tags.reshape(B // TBAGS, TBAGS * TAG_LEN)
    cats_g = categories.reshape(B // CBAGS, CBAGS * CAT_LEN)
    return _encode(tags_g, cats_g, tag_table, cat_table)


# speedup vs baseline: 1.1480x; 1.0010x over previous
import functools

import jax
import jax.numpy as jnp
from jax import lax
from jax.experimental import pallas as pl
from jax.experimental.pallas import tpu as pltpu
from jax.experimental.pallas import tpu_sc as plsc

B = 4096
D = 64
TAG_LEN = 50
CAT_LEN = 20
L = 16
NC = 2
NS = 16
NW = NC * NS
BPW = B // NW

TBAGS = 2
CBAGS = 4
TG = BPW // TBAGS
CG = BPW // CBAGS
TGP = TBAGS * TAG_LEN + 4
CGP = CBAGS * CAT_LEN
NCAT = 1000

_mesh = plsc.VectorSubcoreMesh(core_axis_name="c", subcore_axis_name="s")


@functools.partial(
    pl.kernel,
    mesh=_mesh,
    compiler_params=pltpu.CompilerParams(use_tc_tiling_on_sc=False),
    out_type=(
        jax.ShapeDtypeStruct((B, D), jnp.float32),
        jax.ShapeDtypeStruct((B, D), jnp.float32),
    ),
    scratch_types=[
        pltpu.VMEM((TG * TGP,), jnp.int32),
        pltpu.VMEM((CG * CGP,), jnp.int32),
        pltpu.VMEM((2, TBAGS * TAG_LEN, D), jnp.float32),
        pltpu.VMEM((BPW, D), jnp.float32),
        pltpu.VMEM((BPW, D), jnp.float32),
        pltpu.VMEM_SHARED((NCAT, D), jnp.float32),
        pltpu.SemaphoreType.DMA,
        pltpu.SemaphoreType.DMA,
    ],
)
def _encode(tags_f, cats_f, tag_tab, cat_tab, out_t, out_c,
            tidx, cidx, rows, acc_t, acc_c, cat_sp, sem0, sem1):
    wid = lax.axis_index("s") * NC + lax.axis_index("c")
    b_base = wid * BPW
    sems = (sem0, sem1)

    pltpu.sync_copy(tags_f.at[pl.ds(wid * TG * TGP, TG * TGP)], tidx)
    pltpu.sync_copy(cats_f.at[pl.ds(wid * CG * CGP, CG * CGP)], cidx)

    @pl.when(lax.axis_index("s") == 0)
    def _():
        pltpu.sync_copy(cat_tab, cat_sp)

    def reduce_group(p, g, n_bags, bag_len, acc):
        for q in range(n_bags):
            b = g * n_bags + q
            r0 = q * bag_len
            for d in range(4):
                sl = pl.ds(d * L, L)
                v0 = rows[p, r0, sl]
                v1 = rows[p, r0 + 1, sl]
                for j in range(2, bag_len, 2):
                    v0 = v0 + rows[p, r0 + j, sl]
                    v1 = v1 + rows[p, r0 + j + 1, sl]
                acc[b, sl] = v0 + v1

    def run_phase(idx_ref, stride, tab, n_groups, n_bags, bag_len, acc):
        n_idx = n_bags * bag_len
        dst = lambda p: rows.at[p, pl.ds(0, n_idx)]

        def fire(g, p):
            pltpu.async_copy(tab.at[idx_ref.at[pl.ds(g * stride, n_idx)]],
                             dst(p), sems[p])

        def wait(p):
            pltpu.make_async_copy(tab.at[idx_ref.at[pl.ds(0, n_idx)]],
                                  dst(p), sems[p]).wait()

        fire(0, 0)
        fire(1, 1)

        def body(gg, carry):
            for p in range(2):
                g = 2 * gg + p
                wait(p)
                reduce_group(p, g, n_bags, bag_len, acc)

                @pl.when(g + 2 < n_groups)
                def _():
                    fire(g + 2, p)
            return carry

        lax.fori_loop(0, n_groups // 2, body, 0)

    run_phase(tidx, TGP, tag_tab, TG, TBAGS, TAG_LEN, acc_t)
    plsc.subcore_barrier()
    run_phase(cidx, CGP, cat_sp, CG, CBAGS, CAT_LEN, acc_c)

    pltpu.sync_copy(acc_t, out_t.at[pl.ds(b_base, BPW)])
    pltpu.sync_copy(acc_c, out_c.at[pl.ds(b_base, BPW)])


def kernel(tags, categories, tag_table, cat_table):
    tags_f = jnp.pad(tags.reshape(B // TBAGS, TBAGS * TAG_LEN),
                     ((0, 0), (0, TGP - TBAGS * TAG_LEN))).reshape(-1)
    cats_f = categories.reshape(-1)
    return _encode(tags_f, cats_f, tag_table, cat_table)
